# Initial kernel scaffold; baseline (speedup 1.0000x reference)
import jax
import jax.numpy as jnp
from jax.experimental import pallas as pl


def kernel(vertices, feature_map, pW1, pb1, pg1, pbe1, pW2, pb2, aW1, ab1, ag1, abe1, aW2, ab2):
    raise NotImplementedError("write your pallas kernel here")


# speedup vs baseline: 4.9454x; 4.9454x over previous
import functools

import jax
import jax.numpy as jnp
from jax import lax
from jax.experimental import pallas as pl
from jax.experimental.pallas import tpu as pltpu
from jax.experimental.pallas import tpu_sc as plsc

_B, _N, _C, _NP, _K, _D = 8, 4096, 256, 512, 16, 64
_PW = _C + 16
_L = 16
_NC, _NS = 2, 16
_NW = _NC * _NS
_MB = 32
_RB = _MB * _K


def _fps_body(v_ref, kp_ref, fpsg_ref):
    v = v_ref[...]
    x, y, z = v[:, 0, :], v[:, 1, :], v[:, 2, :]
    iota_n = lax.broadcasted_iota(jnp.int32, (_B, _N), 1)
    iota_p = lax.broadcasted_iota(jnp.int32, (_B, _NP), 1)
    boff = lax.broadcasted_iota(jnp.int32, (_B, 1), 0) * _N

    def body(i, st):
        dists, far, kx, ky, kz, fi = st
        onehot = iota_n == far
        cx = jnp.sum(jnp.where(onehot, x, 0.0), axis=1, keepdims=True)
        cy = jnp.sum(jnp.where(onehot, y, 0.0), axis=1, keepdims=True)
        cz = jnp.sum(jnp.where(onehot, z, 0.0), axis=1, keepdims=True)
        sel = iota_p == i
        kx = jnp.where(sel, jnp.broadcast_to(cx, (_B, _NP)), kx)
        ky = jnp.where(sel, jnp.broadcast_to(cy, (_B, _NP)), ky)
        kz = jnp.where(sel, jnp.broadcast_to(cz, (_B, _NP)), kz)
        fi = jnp.where(sel, jnp.broadcast_to(far, (_B, _NP)), fi)
        dx, dy, dz = x - cx, y - cy, z - cz
        d = (dx * dx + dy * dy) + dz * dz
        dists = jnp.minimum(dists, d)
        m = jnp.max(dists, axis=1, keepdims=True)
        far = jnp.min(jnp.where(dists == m, iota_n, _N), axis=1, keepdims=True)
        return dists, far, kx, ky, kz, fi

    init = (
        x * 0.0 + 1e10,
        jnp.zeros((_B, 1), jnp.int32),
        x[:, :_NP],
        y[:, :_NP],
        z[:, :_NP],
        iota_p + boff,
    )
    _, _, kx, ky, kz, fi = lax.fori_loop(0, _NP, body, init)
    kp_ref[:, 0, :] = kx
    kp_ref[:, 1, :] = ky
    kp_ref[:, 2, :] = kz
    fpsg_ref[...] = fi + boff


@jax.jit
def _fps(vertices):
    return pl.pallas_call(
        _fps_body,
        out_shape=[
            jax.ShapeDtypeStruct((_B, 3, _NP), jnp.float32),
            jax.ShapeDtypeStruct((_B, _NP), jnp.int32),
        ],
    )(vertices)


_MROWS = 256


def _dist_body(kp_ref, v_ref, d_ref):
    kp = kp_ref[0]
    xyz = v_ref[0]
    mm = lax.dot_general(
        kp.astype(jnp.bfloat16), xyz.astype(jnp.bfloat16),
        (((0,), (0,)), ((), ())),
        preferred_element_type=jnp.float32)
    sn = (kp[0] * kp[0] + kp[1] * kp[1]) + kp[2] * kp[2]
    dn = (xyz[0] * xyz[0] + xyz[1] * xyz[1]) + xyz[2] * xyz[2]
    d_ref[0] = (-2.0 * mm + sn[:, None]) + dn[None, :]


@jax.jit
def _dist(key_point, vertices):
    return pl.pallas_call(
        _dist_body,
        grid=(_B, _NP // _MROWS),
        in_specs=[
            pl.BlockSpec((1, 3, _MROWS), lambda b, j: (b, 0, j)),
            pl.BlockSpec((1, 3, _N), lambda b, j: (b, 0, 0)),
        ],
        out_specs=pl.BlockSpec((1, _MROWS, _N), lambda b, j: (b, j, 0)),
        out_shape=jax.ShapeDtypeStruct((_B, _NP, _N), jnp.float32),
    )(key_point, vertices)


_ROWS_W = _B * _NP // _NW
_CHUNKS = _N // _L


def _splat(ch, j):
    return lax.gather(
        ch, jnp.full((_L, 1), j, jnp.int32),
        lax.GatherDimensionNumbers(offset_dims=(), collapsed_slice_dims=(0,),
                                   start_index_map=(0,)),
        slice_sizes=(1,), mode=lax.GatherScatterMode.PROMISE_IN_BOUNDS)


def _knn_body(d_hbm, vert_hbm, kp_hbm, out_hbm, prx_hbm, pry_hbm, prz_hbm,
              xv, yv, zv, kxv, kyv, kzv, idxbuf, pxb, pyb, pzb, dbuf):
    wid = lax.axis_index("s") * _NC + lax.axis_index("c")
    b = wid // (_NW // _B)
    q = wid % (_NW // _B)
    m0 = q * _ROWS_W
    pltpu.sync_copy(vert_hbm.at[pl.ds((b * 3 + 0) * _N, _N)], xv)
    pltpu.sync_copy(vert_hbm.at[pl.ds((b * 3 + 1) * _N, _N)], yv)
    pltpu.sync_copy(vert_hbm.at[pl.ds((b * 3 + 2) * _N, _N)], zv)
    pltpu.sync_copy(kp_hbm.at[pl.ds((b * 3 + 0) * _NP + m0, _ROWS_W)], kxv)
    pltpu.sync_copy(kp_hbm.at[pl.ds((b * 3 + 1) * _NP + m0, _ROWS_W)], kyv)
    pltpu.sync_copy(kp_hbm.at[pl.ds((b * 3 + 2) * _NP + m0, _ROWS_W)], kzv)

    lane = lax.iota(jnp.int32, _L)
    row0 = b * _NP + m0

    def row_loop(r, _):
        g, j = r // _L, r % _L
        kx = _splat(kxv[pl.ds(g * _L, _L)], j)
        ky = _splat(kyv[pl.ds(g * _L, _L)], j)
        kz = _splat(kzv[pl.ds(g * _L, _L)], j)
        pltpu.sync_copy(d_hbm.at[pl.ds((row0 + r) * _N, _N)], dbuf)

        def chunk_loop(t, carry):
            rv, ri, thr = carry
            d = dbuf[pl.ds(t * _L, _L)]
            hit = jnp.any(d < thr)

            def do_merge(args):
                rv_, ri_, _ = args
                sv, si = plsc.sort_key_val(d, lane + t * _L)
                rvr = lax.rev(rv_, (0,))
                rir = lax.rev(ri_, (0,))
                takea = sv <= rvr
                lo_v = jnp.where(takea, sv, rvr)
                lo_i = jnp.where(takea, si, rir)
                nrv, nri = plsc.sort_key_val(lo_v, lo_i)
                return nrv, nri, _splat(nrv, 15)

            return lax.cond(hit, do_merge, lambda a: a, (rv, ri, thr))

        rv0 = jnp.full((_L,), jnp.inf, jnp.float32)
        ri0 = jnp.zeros((_L,), jnp.int32)
        _, ri, _ = lax.fori_loop(0, _CHUNKS, chunk_loop,
                                 (rv0, ri0, rv0))
        idxbuf[pl.ds(r * _K, _K)] = ri + b * _N
        pxb[pl.ds(r * _K, _K)] = kx - plsc.load_gather(xv, [ri])
        pyb[pl.ds(r * _K, _K)] = ky - plsc.load_gather(yv, [ri])
        pzb[pl.ds(r * _K, _K)] = kz - plsc.load_gather(zv, [ri])
        return 0

    lax.fori_loop(0, _ROWS_W, row_loop, 0)
    o0 = (b * _NP + m0) * _K
    pltpu.sync_copy(idxbuf, out_hbm.at[pl.ds(o0, _ROWS_W * _K)])
    pltpu.sync_copy(pxb, prx_hbm.at[pl.ds(o0, _ROWS_W * _K)])
    pltpu.sync_copy(pyb, pry_hbm.at[pl.ds(o0, _ROWS_W * _K)])
    pltpu.sync_copy(pzb, prz_hbm.at[pl.ds(o0, _ROWS_W * _K)])


@jax.jit
def _knn(vertices, key_point):
    f = functools.partial(
        pl.kernel,
        out_type=[
            jax.ShapeDtypeStruct((_B * _NP * _K,), jnp.int32),
            jax.ShapeDtypeStruct((_B * _NP * _K,), jnp.float32),
            jax.ShapeDtypeStruct((_B * _NP * _K,), jnp.float32),
            jax.ShapeDtypeStruct((_B * _NP * _K,), jnp.float32),
        ],
        mesh=plsc.VectorSubcoreMesh(core_axis_name="c", subcore_axis_name="s"),
        compiler_params=pltpu.CompilerParams(needs_layout_passes=False),
        scratch_types=[
            pltpu.VMEM((_N,), jnp.float32),
            pltpu.VMEM((_N,), jnp.float32),
            pltpu.VMEM((_N,), jnp.float32),
            pltpu.VMEM((_ROWS_W,), jnp.float32),
            pltpu.VMEM((_ROWS_W,), jnp.float32),
            pltpu.VMEM((_ROWS_W,), jnp.float32),
            pltpu.VMEM((_ROWS_W * _K,), jnp.int32),
            pltpu.VMEM((_ROWS_W * _K,), jnp.float32),
            pltpu.VMEM((_ROWS_W * _K,), jnp.float32),
            pltpu.VMEM((_ROWS_W * _K,), jnp.float32),
            pltpu.VMEM((_N,), jnp.float32),
        ],
    )(_knn_body)
    d = _dist(key_point, vertices)
    return f(d.reshape(-1), vertices.reshape(-1), key_point.reshape(-1))


_GCH = 128
_GN = _B * _NP * _K // _NW // _GCH


def _gather_body(tab_hbm, gidx_hbm, kidx_hbm, gout_hbm, kout_hbm,
                 idxv, rowsv, sem):
    wid = lax.axis_index("s") * _NC + lax.axis_index("c")
    base = wid * _GN * _GCH

    def chunk(t, _):
        off = base + t * _GCH
        pltpu.sync_copy(gidx_hbm.at[pl.ds(off, _GCH)], idxv)
        cp = pltpu.make_async_copy(tab_hbm.at[idxv], rowsv, sem)
        cp.start()
        cp.wait()
        pltpu.sync_copy(rowsv, gout_hbm.at[pl.ds(off, _GCH)])
        return 0

    lax.fori_loop(0, _GN, chunk, 0)

    kbase = wid * _GCH
    pltpu.sync_copy(kidx_hbm.at[pl.ds(kbase, _GCH)], idxv)
    cp = pltpu.make_async_copy(tab_hbm.at[idxv], rowsv, sem)
    cp.start()
    cp.wait()
    pltpu.sync_copy(rowsv, kout_hbm.at[pl.ds(kbase, _GCH)])


@jax.jit
def _gather(tab, gidx, kidx):
    f = functools.partial(
        pl.kernel,
        out_type=[
            jax.ShapeDtypeStruct((_B * _NP * _K, _C), jnp.float32),
            jax.ShapeDtypeStruct((_B * _NP, _C), jnp.float32),
        ],
        mesh=plsc.VectorSubcoreMesh(core_axis_name="c", subcore_axis_name="s"),
        compiler_params=pltpu.CompilerParams(needs_layout_passes=False),
        scratch_types=[
            pltpu.VMEM((_GCH,), jnp.int32),
            pltpu.VMEM((_GCH, _C), jnp.float32),
            pltpu.SemaphoreType.DMA,
        ],
    )(_gather_body)
    return f(tab, gidx, kidx)


def _mlp_body(g_ref, kf_ref, px_ref, py_ref, pz_ref,
              w1_ref, b1_ref, w2_ref, b2_ref,
              a1_ref, ab1_ref, a2_ref, ab2_ref, out_ref):
    feat = g_ref[...]
    kf = kf_ref[...]
    kfe = jnp.broadcast_to(kf[:, None, :], (_MB, _K, _C)).reshape(_RB, _C)

    dot = functools.partial(jnp.dot, precision=lax.Precision.HIGHEST,
                            preferred_element_type=jnp.float32)
    px, py, pz = px_ref[...], py_ref[...], pz_ref[...]
    h = (px * w1_ref[0:1, :] + py * w1_ref[1:2, :] + pz * w1_ref[2:3, :]
         + b1_ref[...])
    h = jnp.where(h >= 0, h, 0.2 * h)
    pe = dot(h, w2_ref[...]) + b2_ref[...]
    qk = kfe - feat
    a = dot(qk + pe, a1_ref[...]) + ab1_ref[...]
    a = jnp.where(a >= 0, a, 0.2 * a)
    logits = dot(a, a2_ref[...]) + ab2_ref[...]

    l3 = logits.reshape(_MB, _K, _C)
    mx = jnp.max(l3, axis=1, keepdims=True)
    e = jnp.exp(l3 - mx)
    w = e / jnp.sum(e, axis=1, keepdims=True)
    v3 = (feat + pe).reshape(_MB, _K, _C)
    out_ref[...] = jnp.sum(w * v3, axis=1)


@jax.jit
def _mlp(g_rows, k_rows, px, py, pz, w1, b1, w2, b2, a1, ab1, a2, ab2):
    nblk = _B * _NP // _MB
    wspec = lambda shp: pl.BlockSpec(shp, lambda i: (0, 0))
    return pl.pallas_call(
        _mlp_body,
        grid=(nblk,),
        in_specs=[
            pl.BlockSpec((_RB, _C), lambda i: (i, 0)),
            pl.BlockSpec((_MB, _C), lambda i: (i, 0)),
            pl.BlockSpec((_RB, 1), lambda i: (i, 0)),
            pl.BlockSpec((_RB, 1), lambda i: (i, 0)),
            pl.BlockSpec((_RB, 1), lambda i: (i, 0)),
            wspec((8, _D)), wspec((1, _D)),
            wspec((_D, _C)), wspec((1, _C)),
            wspec((_C, _D)), wspec((1, _D)),
            wspec((_D, _C)), wspec((1, _C)),
        ],
        out_specs=pl.BlockSpec((_MB, _C), lambda i: (i, 0)),
        out_shape=jax.ShapeDtypeStruct((_B * _NP, _C), jnp.float32),
    )(g_rows, k_rows, px, py, pz, w1, b1, w2, b2, a1, ab1, a2, ab2)


def kernel(vertices, feature_map, pW1, pb1, pg1, pbe1, pW2, pb2,
           aW1, ab1, ag1, abe1, aW2, ab2):
    key_point, fps_g = _fps(vertices)
    knn_g, prx, pry, prz = _knn(vertices, key_point)

    feat_t = jnp.transpose(feature_map, (0, 2, 1)).reshape(_B * _N, _C)
    g_rows, k_rows = _gather(feat_t, knn_g, fps_g.reshape(-1))

    inv = jnp.float32(1.0) / jnp.sqrt(jnp.float32(1.0 + 1e-5))
    ps, asc = pg1 * inv, ag1 * inv
    w1 = jnp.pad(pW1, ((0, 0), (0, 8 - 3))).T * ps[None, :]
    b1 = (pb1 * ps + pbe1)[None, :]
    w2 = pW2.T
    b2 = pb2[None, :]
    a1 = aW1.T * asc[None, :]
    ab1f = (ab1 * asc + abe1)[None, :]
    a2 = aW2.T
    ab2f = ab2[None, :]

    out = _mlp(g_rows, k_rows, prx[:, None], pry[:, None], prz[:, None],
               w1, b1, w2, b2, a1, ab1f, a2, ab2f)
    new_feat = jnp.transpose(out.reshape(_B, _NP, _C), (0, 2, 1))
    return key_point, new_feat


# speedup vs baseline: 5.8576x; 1.1845x over previous
import functools

import jax
import jax.numpy as jnp
from jax import lax
from jax.experimental import pallas as pl
from jax.experimental.pallas import tpu as pltpu
from jax.experimental.pallas import tpu_sc as plsc

_B, _N, _C, _NP, _K, _D = 8, 4096, 256, 512, 16, 64
_PW = _C + 16
_L = 16
_NC, _NS = 2, 16
_NW = _NC * _NS
_MB = 32
_RB = _MB * _K


def _fps_body(v_ref, kp_ref, fpsg_ref):
    v = v_ref[...]
    x, y, z = v[:, 0, :], v[:, 1, :], v[:, 2, :]
    iota_n = lax.broadcasted_iota(jnp.int32, (_B, _N), 1)
    iota_p = lax.broadcasted_iota(jnp.int32, (_B, _NP), 1)
    boff = lax.broadcasted_iota(jnp.int32, (_B, 1), 0) * _N

    def body(i, st):
        dists, far, kx, ky, kz, fi = st
        onehot = iota_n == far
        cx = jnp.sum(jnp.where(onehot, x, 0.0), axis=1, keepdims=True)
        cy = jnp.sum(jnp.where(onehot, y, 0.0), axis=1, keepdims=True)
        cz = jnp.sum(jnp.where(onehot, z, 0.0), axis=1, keepdims=True)
        sel = iota_p == i
        kx = jnp.where(sel, jnp.broadcast_to(cx, (_B, _NP)), kx)
        ky = jnp.where(sel, jnp.broadcast_to(cy, (_B, _NP)), ky)
        kz = jnp.where(sel, jnp.broadcast_to(cz, (_B, _NP)), kz)
        fi = jnp.where(sel, jnp.broadcast_to(far, (_B, _NP)), fi)
        dx, dy, dz = x - cx, y - cy, z - cz
        d = (dx * dx + dy * dy) + dz * dz
        dists = jnp.minimum(dists, d)
        m = jnp.max(dists, axis=1, keepdims=True)
        far = jnp.min(jnp.where(dists == m, iota_n, _N), axis=1, keepdims=True)
        return dists, far, kx, ky, kz, fi

    init = (
        x * 0.0 + 1e10,
        jnp.zeros((_B, 1), jnp.int32),
        x[:, :_NP],
        y[:, :_NP],
        z[:, :_NP],
        iota_p + boff,
    )
    _, _, kx, ky, kz, fi = lax.fori_loop(0, _NP, body, init)
    kp_ref[:, 0, :] = kx
    kp_ref[:, 1, :] = ky
    kp_ref[:, 2, :] = kz
    fpsg_ref[...] = fi + boff


@jax.jit
def _fps(vertices):
    return pl.pallas_call(
        _fps_body,
        out_shape=[
            jax.ShapeDtypeStruct((_B, 3, _NP), jnp.float32),
            jax.ShapeDtypeStruct((_B, _NP), jnp.int32),
        ],
    )(vertices)


_MROWS = 256


def _dist_body(kp_ref, v_ref, d_ref):
    kp = kp_ref[0]
    xyz = v_ref[0]
    mm = lax.dot_general(
        kp.astype(jnp.bfloat16), xyz.astype(jnp.bfloat16),
        (((0,), (0,)), ((), ())),
        preferred_element_type=jnp.float32)
    sn = (kp[0] * kp[0] + kp[1] * kp[1]) + kp[2] * kp[2]
    dn = (xyz[0] * xyz[0] + xyz[1] * xyz[1]) + xyz[2] * xyz[2]
    d_ref[0] = (-2.0 * mm + sn[:, None]) + dn[None, :]


@jax.jit
def _dist(key_point, vertices):
    return pl.pallas_call(
        _dist_body,
        grid=(_B, _NP // _MROWS),
        in_specs=[
            pl.BlockSpec((1, 3, _MROWS), lambda b, j: (b, 0, j)),
            pl.BlockSpec((1, 3, _N), lambda b, j: (b, 0, 0)),
        ],
        out_specs=pl.BlockSpec((1, _MROWS, _N), lambda b, j: (b, j, 0)),
        out_shape=jax.ShapeDtypeStruct((_B, _NP, _N), jnp.float32),
    )(key_point, vertices)


_ROWS_W = _B * _NP // _NW
_CHUNKS = _N // _L


def _splat(ch, j):
    return lax.gather(
        ch, jnp.full((_L, 1), j, jnp.int32),
        lax.GatherDimensionNumbers(offset_dims=(), collapsed_slice_dims=(0,),
                                   start_index_map=(0,)),
        slice_sizes=(1,), mode=lax.GatherScatterMode.PROMISE_IN_BOUNDS)


_G = 8
_NG = _CHUNKS // _G


def _knn_body(d_hbm, vert_hbm, kp_hbm, out_hbm, prx_hbm, pry_hbm, prz_hbm,
              xv, yv, zv, kxv, kyv, kzv, idxbuf, pxb, pyb, pzb,
              db0, db1, sem0, sem1):
    wid = lax.axis_index("s") * _NC + lax.axis_index("c")
    b = wid // (_NW // _B)
    q = wid % (_NW // _B)
    m0 = q * _ROWS_W
    pltpu.sync_copy(vert_hbm.at[pl.ds((b * 3 + 0) * _N, _N)], xv)
    pltpu.sync_copy(vert_hbm.at[pl.ds((b * 3 + 1) * _N, _N)], yv)
    pltpu.sync_copy(vert_hbm.at[pl.ds((b * 3 + 2) * _N, _N)], zv)
    pltpu.sync_copy(kp_hbm.at[pl.ds((b * 3 + 0) * _NP + m0, _ROWS_W)], kxv)
    pltpu.sync_copy(kp_hbm.at[pl.ds((b * 3 + 1) * _NP + m0, _ROWS_W)], kyv)
    pltpu.sync_copy(kp_hbm.at[pl.ds((b * 3 + 2) * _NP + m0, _ROWS_W)], kzv)

    lane = lax.iota(jnp.int32, _L)
    row0 = b * _NP + m0

    def chunk_merge(d, cidx, carry):
        rv, ri, thr = carry
        hit = jnp.any(d < thr)

        def do_merge(args):
            rv_, ri_, _ = args
            sv, si = plsc.sort_key_val(d, cidx)
            rvr = lax.rev(rv_, (0,))
            rir = lax.rev(ri_, (0,))
            takea = sv < rvr
            lo_v = jnp.where(takea, sv, rvr)
            lo_i = jnp.where(takea, si, rir)
            nrv, nri = plsc.sort_key_val(lo_v, lo_i)
            return nrv, nri, _splat(nrv, 15)

        return lax.cond(hit, do_merge, lambda a: a, (rv, ri, thr))

    def do_row(r, dbuf):
        g, j = r // _L, r % _L
        kx = _splat(kxv[pl.ds(g * _L, _L)], j)
        ky = _splat(kyv[pl.ds(g * _L, _L)], j)
        kz = _splat(kzv[pl.ds(g * _L, _L)], j)

        def group_loop(t, carry):
            rv, ri, thr = carry
            base = t * _G * _L
            ds_ = [dbuf[pl.ds(base + u * _L, _L)] for u in range(_G)]
            gmin = ds_[0]
            for u in range(1, _G):
                gmin = jnp.minimum(gmin, ds_[u])
            ghit = jnp.any(gmin < thr)

            def scan_group(args):
                c = args
                for u in range(_G):
                    c = chunk_merge(ds_[u], lane + (base + u * _L), c)
                return c

            return lax.cond(ghit, scan_group, lambda a: a, (rv, ri, thr))

        rv0 = jnp.full((_L,), jnp.inf, jnp.float32)
        ri0 = jnp.zeros((_L,), jnp.int32)
        _, ri, _ = lax.fori_loop(0, _NG, group_loop, (rv0, ri0, rv0))
        idxbuf[pl.ds(r * _K, _K)] = ri + b * _N
        pxb[pl.ds(r * _K, _K)] = kx - plsc.load_gather(xv, [ri])
        pyb[pl.ds(r * _K, _K)] = ky - plsc.load_gather(yv, [ri])
        pzb[pl.ds(r * _K, _K)] = kz - plsc.load_gather(zv, [ri])

    pltpu.make_async_copy(d_hbm.at[pl.ds(row0 * _N, _N)], db0, sem0).start()

    def pair_loop(p, _):
        for par in range(2):
            r = 2 * p + par
            cur, csem = (db0, sem0) if par == 0 else (db1, sem1)
            nxt, nsem = (db1, sem1) if par == 0 else (db0, sem0)
            nr = jnp.minimum(r + 1, _ROWS_W - 1)
            pltpu.make_async_copy(
                d_hbm.at[pl.ds((row0 + nr) * _N, _N)], nxt, nsem).start()
            pltpu.make_async_copy(
                d_hbm.at[pl.ds((row0 + r) * _N, _N)], cur, csem).wait()
            do_row(r, cur)
        return 0

    lax.fori_loop(0, _ROWS_W // 2, pair_loop, 0)
    pltpu.make_async_copy(d_hbm.at[pl.ds(row0 * _N, _N)], db0, sem0).wait()

    o0 = (b * _NP + m0) * _K
    pltpu.sync_copy(idxbuf, out_hbm.at[pl.ds(o0, _ROWS_W * _K)])
    pltpu.sync_copy(pxb, prx_hbm.at[pl.ds(o0, _ROWS_W * _K)])
    pltpu.sync_copy(pyb, pry_hbm.at[pl.ds(o0, _ROWS_W * _K)])
    pltpu.sync_copy(pzb, prz_hbm.at[pl.ds(o0, _ROWS_W * _K)])


@jax.jit
def _knn(vertices, key_point):
    f = functools.partial(
        pl.kernel,
        out_type=[
            jax.ShapeDtypeStruct((_B * _NP * _K,), jnp.int32),
            jax.ShapeDtypeStruct((_B * _NP * _K,), jnp.float32),
            jax.ShapeDtypeStruct((_B * _NP * _K,), jnp.float32),
            jax.ShapeDtypeStruct((_B * _NP * _K,), jnp.float32),
        ],
        mesh=plsc.VectorSubcoreMesh(core_axis_name="c", subcore_axis_name="s"),
        compiler_params=pltpu.CompilerParams(needs_layout_passes=False),
        scratch_types=[
            pltpu.VMEM((_N,), jnp.float32),
            pltpu.VMEM((_N,), jnp.float32),
            pltpu.VMEM((_N,), jnp.float32),
            pltpu.VMEM((_ROWS_W,), jnp.float32),
            pltpu.VMEM((_ROWS_W,), jnp.float32),
            pltpu.VMEM((_ROWS_W,), jnp.float32),
            pltpu.VMEM((_ROWS_W * _K,), jnp.int32),
            pltpu.VMEM((_ROWS_W * _K,), jnp.float32),
            pltpu.VMEM((_ROWS_W * _K,), jnp.float32),
            pltpu.VMEM((_ROWS_W * _K,), jnp.float32),
            pltpu.VMEM((_N,), jnp.float32),
            pltpu.VMEM((_N,), jnp.float32),
            pltpu.SemaphoreType.DMA,
            pltpu.SemaphoreType.DMA,
        ],
    )(_knn_body)
    d = _dist(key_point, vertices)
    return f(d.reshape(-1), vertices.reshape(-1), key_point.reshape(-1))


_GCH = 128
_GN = _B * _NP * _K // _NW // _GCH


def _gather_body(tab_hbm, gidx_hbm, kidx_hbm, gout_hbm, kout_hbm,
                 idxv, rowsv, sem):
    wid = lax.axis_index("s") * _NC + lax.axis_index("c")
    base = wid * _GN * _GCH

    def chunk(t, _):
        off = base + t * _GCH
        pltpu.sync_copy(gidx_hbm.at[pl.ds(off, _GCH)], idxv)
        cp = pltpu.make_async_copy(tab_hbm.at[idxv], rowsv, sem)
        cp.start()
        cp.wait()
        pltpu.sync_copy(rowsv, gout_hbm.at[pl.ds(off, _GCH)])
        return 0

    lax.fori_loop(0, _GN, chunk, 0)

    kbase = wid * _GCH
    pltpu.sync_copy(kidx_hbm.at[pl.ds(kbase, _GCH)], idxv)
    cp = pltpu.make_async_copy(tab_hbm.at[idxv], rowsv, sem)
    cp.start()
    cp.wait()
    pltpu.sync_copy(rowsv, kout_hbm.at[pl.ds(kbase, _GCH)])


@jax.jit
def _gather(tab, gidx, kidx):
    f = functools.partial(
        pl.kernel,
        out_type=[
            jax.ShapeDtypeStruct((_B * _NP * _K, _C), jnp.float32),
            jax.ShapeDtypeStruct((_B * _NP, _C), jnp.float32),
        ],
        mesh=plsc.VectorSubcoreMesh(core_axis_name="c", subcore_axis_name="s"),
        compiler_params=pltpu.CompilerParams(needs_layout_passes=False),
        scratch_types=[
            pltpu.VMEM((_GCH,), jnp.int32),
            pltpu.VMEM((_GCH, _C), jnp.float32),
            pltpu.SemaphoreType.DMA,
        ],
    )(_gather_body)
    return f(tab, gidx, kidx)


def _mlp_body(g_ref, kf_ref, px_ref, py_ref, pz_ref,
              w1_ref, b1_ref, w2_ref, b2_ref,
              a1_ref, ab1_ref, a2_ref, ab2_ref, out_ref):
    feat = g_ref[...]
    kf = kf_ref[...]
    kfe = jnp.broadcast_to(kf[:, None, :], (_MB, _K, _C)).reshape(_RB, _C)

    dot = functools.partial(jnp.dot, precision=lax.Precision.HIGHEST,
                            preferred_element_type=jnp.float32)
    px, py, pz = px_ref[...], py_ref[...], pz_ref[...]
    h = (px * w1_ref[0:1, :] + py * w1_ref[1:2, :] + pz * w1_ref[2:3, :]
         + b1_ref[...])
    h = jnp.where(h >= 0, h, 0.2 * h)
    pe = dot(h, w2_ref[...]) + b2_ref[...]
    qk = kfe - feat
    a = dot(qk + pe, a1_ref[...]) + ab1_ref[...]
    a = jnp.where(a >= 0, a, 0.2 * a)
    logits = dot(a, a2_ref[...]) + ab2_ref[...]

    l3 = logits.reshape(_MB, _K, _C)
    mx = jnp.max(l3, axis=1, keepdims=True)
    e = jnp.exp(l3 - mx)
    w = e / jnp.sum(e, axis=1, keepdims=True)
    v3 = (feat + pe).reshape(_MB, _K, _C)
    out_ref[...] = jnp.sum(w * v3, axis=1)


@jax.jit
def _mlp(g_rows, k_rows, px, py, pz, w1, b1, w2, b2, a1, ab1, a2, ab2):
    nblk = _B * _NP // _MB
    wspec = lambda shp: pl.BlockSpec(shp, lambda i: (0, 0))
    return pl.pallas_call(
        _mlp_body,
        grid=(nblk,),
        in_specs=[
            pl.BlockSpec((_RB, _C), lambda i: (i, 0)),
            pl.BlockSpec((_MB, _C), lambda i: (i, 0)),
            pl.BlockSpec((_RB, 1), lambda i: (i, 0)),
            pl.BlockSpec((_RB, 1), lambda i: (i, 0)),
            pl.BlockSpec((_RB, 1), lambda i: (i, 0)),
            wspec((8, _D)), wspec((1, _D)),
            wspec((_D, _C)), wspec((1, _C)),
            wspec((_C, _D)), wspec((1, _D)),
            wspec((_D, _C)), wspec((1, _C)),
        ],
        out_specs=pl.BlockSpec((_MB, _C), lambda i: (i, 0)),
        out_shape=jax.ShapeDtypeStruct((_B * _NP, _C), jnp.float32),
    )(g_rows, k_rows, px, py, pz, w1, b1, w2, b2, a1, ab1, a2, ab2)


def kernel(vertices, feature_map, pW1, pb1, pg1, pbe1, pW2, pb2,
           aW1, ab1, ag1, abe1, aW2, ab2):
    key_point, fps_g = _fps(vertices)
    knn_g, prx, pry, prz = _knn(vertices, key_point)

    feat_t = jnp.transpose(feature_map, (0, 2, 1)).reshape(_B * _N, _C)
    g_rows, k_rows = _gather(feat_t, knn_g, fps_g.reshape(-1))

    inv = jnp.float32(1.0) / jnp.sqrt(jnp.float32(1.0 + 1e-5))
    ps, asc = pg1 * inv, ag1 * inv
    w1 = jnp.pad(pW1, ((0, 0), (0, 8 - 3))).T * ps[None, :]
    b1 = (pb1 * ps + pbe1)[None, :]
    w2 = pW2.T
    b2 = pb2[None, :]
    a1 = aW1.T * asc[None, :]
    ab1f = (ab1 * asc + abe1)[None, :]
    a2 = aW2.T
    ab2f = ab2[None, :]

    out = _mlp(g_rows, k_rows, prx[:, None], pry[:, None], prz[:, None],
               w1, b1, w2, b2, a1, ab1f, a2, ab2f)
    new_feat = jnp.transpose(out.reshape(_B, _NP, _C), (0, 2, 1))
    return key_point, new_feat


# speedup vs baseline: 5.9272x; 1.0119x over previous
import functools

import jax
import jax.numpy as jnp
from jax import lax
from jax.experimental import pallas as pl
from jax.experimental.pallas import tpu as pltpu
from jax.experimental.pallas import tpu_sc as plsc

_B, _N, _C, _NP, _K, _D = 8, 4096, 256, 512, 16, 64
_PW = _C + 16
_L = 16
_NC, _NS = 2, 16
_NW = _NC * _NS
_MB = 32
_RB = _MB * _K


def _fps_body(v_ref, kp_ref, fpsg_ref):
    v = v_ref[...]
    x, y, z = v[:, 0, :], v[:, 1, :], v[:, 2, :]
    iota_n = lax.broadcasted_iota(jnp.int32, (_B, _N), 1)
    iota_p = lax.broadcasted_iota(jnp.int32, (_B, _NP), 1)
    boff = lax.broadcasted_iota(jnp.int32, (_B, 1), 0) * _N

    def body(i, st):
        dists, far, kx, ky, kz, fi = st
        onehot = iota_n == far
        cx = jnp.sum(jnp.where(onehot, x, 0.0), axis=1, keepdims=True)
        cy = jnp.sum(jnp.where(onehot, y, 0.0), axis=1, keepdims=True)
        cz = jnp.sum(jnp.where(onehot, z, 0.0), axis=1, keepdims=True)
        sel = iota_p == i
        kx = jnp.where(sel, jnp.broadcast_to(cx, (_B, _NP)), kx)
        ky = jnp.where(sel, jnp.broadcast_to(cy, (_B, _NP)), ky)
        kz = jnp.where(sel, jnp.broadcast_to(cz, (_B, _NP)), kz)
        fi = jnp.where(sel, jnp.broadcast_to(far, (_B, _NP)), fi)
        dx, dy, dz = x - cx, y - cy, z - cz
        d = (dx * dx + dy * dy) + dz * dz
        dists = jnp.minimum(dists, d)
        m = jnp.max(dists, axis=1, keepdims=True)
        far = jnp.min(jnp.where(dists == m, iota_n, _N), axis=1, keepdims=True)
        return dists, far, kx, ky, kz, fi

    init = (
        x * 0.0 + 1e10,
        jnp.zeros((_B, 1), jnp.int32),
        x[:, :_NP],
        y[:, :_NP],
        z[:, :_NP],
        iota_p + boff,
    )
    _, _, kx, ky, kz, fi = lax.fori_loop(0, _NP, body, init)
    kp_ref[:, 0, :] = kx
    kp_ref[:, 1, :] = ky
    kp_ref[:, 2, :] = kz
    fpsg_ref[...] = fi + boff


@jax.jit
def _fps(vertices):
    return pl.pallas_call(
        _fps_body,
        out_shape=[
            jax.ShapeDtypeStruct((_B, 3, _NP), jnp.float32),
            jax.ShapeDtypeStruct((_B, _NP), jnp.int32),
        ],
    )(vertices)


_MROWS = 256


def _dist_body(kp_ref, v_ref, d_ref):
    kp = kp_ref[0]
    xyz = v_ref[0]
    mm = lax.dot_general(
        kp.astype(jnp.bfloat16), xyz.astype(jnp.bfloat16),
        (((0,), (0,)), ((), ())),
        preferred_element_type=jnp.float32)
    sn = (kp[0] * kp[0] + kp[1] * kp[1]) + kp[2] * kp[2]
    dn = (xyz[0] * xyz[0] + xyz[1] * xyz[1]) + xyz[2] * xyz[2]
    d_ref[0] = (-2.0 * mm + sn[:, None]) + dn[None, :]


@jax.jit
def _dist(key_point, vertices):
    return pl.pallas_call(
        _dist_body,
        grid=(_B, _NP // _MROWS),
        in_specs=[
            pl.BlockSpec((1, 3, _MROWS), lambda b, j: (b, 0, j)),
            pl.BlockSpec((1, 3, _N), lambda b, j: (b, 0, 0)),
        ],
        out_specs=pl.BlockSpec((1, _MROWS, _N), lambda b, j: (b, j, 0)),
        out_shape=jax.ShapeDtypeStruct((_B, _NP, _N), jnp.float32),
    )(key_point, vertices)


_ROWS_W = _B * _NP // _NW
_CHUNKS = _N // _L


def _splat(ch, j):
    return lax.gather(
        ch, jnp.full((_L, 1), j, jnp.int32),
        lax.GatherDimensionNumbers(offset_dims=(), collapsed_slice_dims=(0,),
                                   start_index_map=(0,)),
        slice_sizes=(1,), mode=lax.GatherScatterMode.PROMISE_IN_BOUNDS)


_G = 16
_NG = _CHUNKS // _G


def _knn_body(d_hbm, vert_hbm, kp_hbm, out_hbm, prx_hbm, pry_hbm, prz_hbm,
              xv, yv, zv, kxv, kyv, kzv, idxbuf, pxb, pyb, pzb,
              db0, db1, sem0, sem1):
    wid = lax.axis_index("s") * _NC + lax.axis_index("c")
    b = wid // (_NW // _B)
    q = wid % (_NW // _B)
    m0 = q * _ROWS_W
    pltpu.sync_copy(vert_hbm.at[pl.ds((b * 3 + 0) * _N, _N)], xv)
    pltpu.sync_copy(vert_hbm.at[pl.ds((b * 3 + 1) * _N, _N)], yv)
    pltpu.sync_copy(vert_hbm.at[pl.ds((b * 3 + 2) * _N, _N)], zv)
    pltpu.sync_copy(kp_hbm.at[pl.ds((b * 3 + 0) * _NP + m0, _ROWS_W)], kxv)
    pltpu.sync_copy(kp_hbm.at[pl.ds((b * 3 + 1) * _NP + m0, _ROWS_W)], kyv)
    pltpu.sync_copy(kp_hbm.at[pl.ds((b * 3 + 2) * _NP + m0, _ROWS_W)], kzv)

    lane = lax.iota(jnp.int32, _L)
    row0 = b * _NP + m0

    def chunk_merge(d, cidx, carry):
        rv, ri, thr = carry
        hit = jnp.any(d < thr)

        def do_merge(args):
            rv_, ri_, _ = args
            sv, si = plsc.sort_key_val(d, cidx)
            rvr = lax.rev(rv_, (0,))
            rir = lax.rev(ri_, (0,))
            takea = sv < rvr
            lo_v = jnp.where(takea, sv, rvr)
            lo_i = jnp.where(takea, si, rir)
            nrv, nri = plsc.sort_key_val(lo_v, lo_i)
            return nrv, nri, _splat(nrv, 15)

        return lax.cond(hit, do_merge, lambda a: a, (rv, ri, thr))

    def do_row(r, dbuf):
        g, j = r // _L, r % _L
        kx = _splat(kxv[pl.ds(g * _L, _L)], j)
        ky = _splat(kyv[pl.ds(g * _L, _L)], j)
        kz = _splat(kzv[pl.ds(g * _L, _L)], j)

        def group_loop(t, carry):
            rv, ri, thr = carry
            base = t * _G * _L
            ds_ = [dbuf[pl.ds(base + u * _L, _L)] for u in range(_G)]
            gmin = ds_[0]
            for u in range(1, _G):
                gmin = jnp.minimum(gmin, ds_[u])
            ghit = jnp.any(gmin < thr)

            def scan_group(args):
                c = args
                for u in range(_G):
                    c = chunk_merge(ds_[u], lane + (base + u * _L), c)
                return c

            return lax.cond(ghit, scan_group, lambda a: a, (rv, ri, thr))

        rv0 = jnp.full((_L,), jnp.inf, jnp.float32)
        ri0 = jnp.zeros((_L,), jnp.int32)
        _, ri, _ = lax.fori_loop(0, _NG, group_loop, (rv0, ri0, rv0))
        idxbuf[pl.ds(r * _K, _K)] = ri + b * _N
        pxb[pl.ds(r * _K, _K)] = kx - plsc.load_gather(xv, [ri])
        pyb[pl.ds(r * _K, _K)] = ky - plsc.load_gather(yv, [ri])
        pzb[pl.ds(r * _K, _K)] = kz - plsc.load_gather(zv, [ri])

    pltpu.make_async_copy(d_hbm.at[pl.ds(row0 * _N, _N)], db0, sem0).start()

    def pair_loop(p, _):
        for par in range(2):
            r = 2 * p + par
            cur, csem = (db0, sem0) if par == 0 else (db1, sem1)
            nxt, nsem = (db1, sem1) if par == 0 else (db0, sem0)
            nr = jnp.minimum(r + 1, _ROWS_W - 1)
            pltpu.make_async_copy(
                d_hbm.at[pl.ds((row0 + nr) * _N, _N)], nxt, nsem).start()
            pltpu.make_async_copy(
                d_hbm.at[pl.ds((row0 + r) * _N, _N)], cur, csem).wait()
            do_row(r, cur)
        return 0

    lax.fori_loop(0, _ROWS_W // 2, pair_loop, 0)
    pltpu.make_async_copy(d_hbm.at[pl.ds(row0 * _N, _N)], db0, sem0).wait()

    o0 = (b * _NP + m0) * _K
    pltpu.sync_copy(idxbuf, out_hbm.at[pl.ds(o0, _ROWS_W * _K)])
    pltpu.sync_copy(pxb, prx_hbm.at[pl.ds(o0, _ROWS_W * _K)])
    pltpu.sync_copy(pyb, pry_hbm.at[pl.ds(o0, _ROWS_W * _K)])
    pltpu.sync_copy(pzb, prz_hbm.at[pl.ds(o0, _ROWS_W * _K)])


@jax.jit
def _knn(vertices, key_point):
    f = functools.partial(
        pl.kernel,
        out_type=[
            jax.ShapeDtypeStruct((_B * _NP * _K,), jnp.int32),
            jax.ShapeDtypeStruct((_B * _NP * _K,), jnp.float32),
            jax.ShapeDtypeStruct((_B * _NP * _K,), jnp.float32),
            jax.ShapeDtypeStruct((_B * _NP * _K,), jnp.float32),
        ],
        mesh=plsc.VectorSubcoreMesh(core_axis_name="c", subcore_axis_name="s"),
        compiler_params=pltpu.CompilerParams(needs_layout_passes=False),
        scratch_types=[
            pltpu.VMEM((_N,), jnp.float32),
            pltpu.VMEM((_N,), jnp.float32),
            pltpu.VMEM((_N,), jnp.float32),
            pltpu.VMEM((_ROWS_W,), jnp.float32),
            pltpu.VMEM((_ROWS_W,), jnp.float32),
            pltpu.VMEM((_ROWS_W,), jnp.float32),
            pltpu.VMEM((_ROWS_W * _K,), jnp.int32),
            pltpu.VMEM((_ROWS_W * _K,), jnp.float32),
            pltpu.VMEM((_ROWS_W * _K,), jnp.float32),
            pltpu.VMEM((_ROWS_W * _K,), jnp.float32),
            pltpu.VMEM((_N,), jnp.float32),
            pltpu.VMEM((_N,), jnp.float32),
            pltpu.SemaphoreType.DMA,
            pltpu.SemaphoreType.DMA,
        ],
    )(_knn_body)
    d = _dist(key_point, vertices)
    return f(d.reshape(-1), vertices.reshape(-1), key_point.reshape(-1))


_GCH = 128
_GN = _B * _NP * _K // _NW // _GCH


def _gather_body(tab_hbm, gidx_hbm, kidx_hbm, gout_hbm, kout_hbm,
                 idxv, rowsv, sem):
    wid = lax.axis_index("s") * _NC + lax.axis_index("c")
    base = wid * _GN * _GCH

    def chunk(t, _):
        off = base + t * _GCH
        pltpu.sync_copy(gidx_hbm.at[pl.ds(off, _GCH)], idxv)
        cp = pltpu.make_async_copy(tab_hbm.at[idxv], rowsv, sem)
        cp.start()
        cp.wait()
        pltpu.sync_copy(rowsv, gout_hbm.at[pl.ds(off, _GCH)])
        return 0

    lax.fori_loop(0, _GN, chunk, 0)

    kbase = wid * _GCH
    pltpu.sync_copy(kidx_hbm.at[pl.ds(kbase, _GCH)], idxv)
    cp = pltpu.make_async_copy(tab_hbm.at[idxv], rowsv, sem)
    cp.start()
    cp.wait()
    pltpu.sync_copy(rowsv, kout_hbm.at[pl.ds(kbase, _GCH)])


@jax.jit
def _gather(tab, gidx, kidx):
    f = functools.partial(
        pl.kernel,
        out_type=[
            jax.ShapeDtypeStruct((_B * _NP * _K, _C), jnp.float32),
            jax.ShapeDtypeStruct((_B * _NP, _C), jnp.float32),
        ],
        mesh=plsc.VectorSubcoreMesh(core_axis_name="c", subcore_axis_name="s"),
        compiler_params=pltpu.CompilerParams(needs_layout_passes=False),
        scratch_types=[
            pltpu.VMEM((_GCH,), jnp.int32),
            pltpu.VMEM((_GCH, _C), jnp.float32),
            pltpu.SemaphoreType.DMA,
        ],
    )(_gather_body)
    return f(tab, gidx, kidx)


def _mlp_body(g_ref, kf_ref, px_ref, py_ref, pz_ref,
              w1_ref, b1_ref, w2_ref, b2_ref,
              a1_ref, ab1_ref, a2_ref, ab2_ref, out_ref):
    feat = g_ref[...]
    kf = kf_ref[...]
    kfe = jnp.broadcast_to(kf[:, None, :], (_MB, _K, _C)).reshape(_RB, _C)

    dot = functools.partial(jnp.dot, preferred_element_type=jnp.float32)
    px, py, pz = px_ref[...], py_ref[...], pz_ref[...]
    h = (px * w1_ref[0:1, :] + py * w1_ref[1:2, :] + pz * w1_ref[2:3, :]
         + b1_ref[...])
    h = jnp.where(h >= 0, h, 0.2 * h)
    pe = dot(h, w2_ref[...]) + b2_ref[...]
    qk = kfe - feat
    a = dot(qk + pe, a1_ref[...]) + ab1_ref[...]
    a = jnp.where(a >= 0, a, 0.2 * a)
    logits = dot(a, a2_ref[...]) + ab2_ref[...]

    l3 = logits.reshape(_MB, _K, _C)
    mx = jnp.max(l3, axis=1, keepdims=True)
    e = jnp.exp(l3 - mx)
    w = e / jnp.sum(e, axis=1, keepdims=True)
    v3 = (feat + pe).reshape(_MB, _K, _C)
    out_ref[...] = jnp.sum(w * v3, axis=1)


@jax.jit
def _mlp(g_rows, k_rows, px, py, pz, w1, b1, w2, b2, a1, ab1, a2, ab2):
    nblk = _B * _NP // _MB
    wspec = lambda shp: pl.BlockSpec(shp, lambda i: (0, 0))
    return pl.pallas_call(
        _mlp_body,
        grid=(nblk,),
        in_specs=[
            pl.BlockSpec((_RB, _C), lambda i: (i, 0)),
            pl.BlockSpec((_MB, _C), lambda i: (i, 0)),
            pl.BlockSpec((_RB, 1), lambda i: (i, 0)),
            pl.BlockSpec((_RB, 1), lambda i: (i, 0)),
            pl.BlockSpec((_RB, 1), lambda i: (i, 0)),
            wspec((8, _D)), wspec((1, _D)),
            wspec((_D, _C)), wspec((1, _C)),
            wspec((_C, _D)), wspec((1, _D)),
            wspec((_D, _C)), wspec((1, _C)),
        ],
        out_specs=pl.BlockSpec((_MB, _C), lambda i: (i, 0)),
        out_shape=jax.ShapeDtypeStruct((_B * _NP, _C), jnp.float32),
    )(g_rows, k_rows, px, py, pz, w1, b1, w2, b2, a1, ab1, a2, ab2)


def kernel(vertices, feature_map, pW1, pb1, pg1, pbe1, pW2, pb2,
           aW1, ab1, ag1, abe1, aW2, ab2):
    key_point, fps_g = _fps(vertices)
    knn_g, prx, pry, prz = _knn(vertices, key_point)

    feat_t = jnp.transpose(feature_map, (0, 2, 1)).reshape(_B * _N, _C)
    g_rows, k_rows = _gather(feat_t, knn_g, fps_g.reshape(-1))

    inv = jnp.float32(1.0) / jnp.sqrt(jnp.float32(1.0 + 1e-5))
    ps, asc = pg1 * inv, ag1 * inv
    w1 = jnp.pad(pW1, ((0, 0), (0, 8 - 3))).T * ps[None, :]
    b1 = (pb1 * ps + pbe1)[None, :]
    w2 = pW2.T
    b2 = pb2[None, :]
    a1 = aW1.T * asc[None, :]
    ab1f = (ab1 * asc + abe1)[None, :]
    a2 = aW2.T
    ab2f = ab2[None, :]

    out = _mlp(g_rows, k_rows, prx[:, None], pry[:, None], prz[:, None],
               w1, b1, w2, b2, a1, ab1f, a2, ab2f)
    new_feat = jnp.transpose(out.reshape(_B, _NP, _C), (0, 2, 1))
    return key_point, new_feat


# speedup vs baseline: 6.9761x; 1.1770x over previous
import functools

import jax
import jax.numpy as jnp
from jax import lax
from jax.experimental import pallas as pl
from jax.experimental.pallas import tpu as pltpu
from jax.experimental.pallas import tpu_sc as plsc

_B, _N, _C, _NP, _K, _D = 8, 4096, 256, 512, 16, 64
_PW = _C + 16
_L = 16
_NC, _NS = 2, 16
_NW = _NC * _NS
_MB = 32
_RB = _MB * _K


def _fps_body(v_ref, kp_ref, fpsg_ref, dist_ref):
    iota_p = lax.broadcasted_iota(jnp.int32, (_B, _NP), 1)
    boff = lax.broadcasted_iota(jnp.int32, (_B, 1), 0) * _N
    dist_ref[...] = v_ref[:, 0, :] * 0.0 + 1e10

    def body(i, st):
        far, kx, ky, kz, fi = st
        iota_n = lax.broadcasted_iota(jnp.int32, (_B, _N), 1)
        onehot = iota_n == far
        x = v_ref[:, 0, :]
        y = v_ref[:, 1, :]
        z = v_ref[:, 2, :]
        cx = jnp.sum(jnp.where(onehot, x, 0.0), axis=1, keepdims=True)
        cy = jnp.sum(jnp.where(onehot, y, 0.0), axis=1, keepdims=True)
        cz = jnp.sum(jnp.where(onehot, z, 0.0), axis=1, keepdims=True)
        sel = iota_p == i
        kx = jnp.where(sel, jnp.broadcast_to(cx, (_B, _NP)), kx)
        ky = jnp.where(sel, jnp.broadcast_to(cy, (_B, _NP)), ky)
        kz = jnp.where(sel, jnp.broadcast_to(cz, (_B, _NP)), kz)
        fi = jnp.where(sel, jnp.broadcast_to(far, (_B, _NP)), fi)
        dx, dy, dz = x - cx, y - cy, z - cz
        d = (dx * dx + dy * dy) + dz * dz
        dists = jnp.minimum(dist_ref[...], d)
        dist_ref[...] = dists
        m = jnp.max(dists, axis=1, keepdims=True)
        far = jnp.min(jnp.where(dists == m, iota_n, _N), axis=1, keepdims=True)
        return far, kx, ky, kz, fi

    init = (
        jnp.zeros((_B, 1), jnp.int32),
        v_ref[:, 0, :_NP],
        v_ref[:, 1, :_NP],
        v_ref[:, 2, :_NP],
        iota_p + boff,
    )
    _, kx, ky, kz, fi = lax.fori_loop(0, _NP, body, init)
    kp_ref[:, 0, :] = kx
    kp_ref[:, 1, :] = ky
    kp_ref[:, 2, :] = kz
    fpsg_ref[...] = fi + boff


@jax.jit
def _fps(vertices):
    return pl.pallas_call(
        _fps_body,
        out_shape=[
            jax.ShapeDtypeStruct((_B, 3, _NP), jnp.float32),
            jax.ShapeDtypeStruct((_B, _NP), jnp.int32),
        ],
        scratch_shapes=[pltpu.VMEM((_B, _N), jnp.float32)],
    )(vertices)


_MROWS = 256


def _dist_body(kp_ref, v_ref, d_ref):
    kp = kp_ref[0]
    xyz = v_ref[0]
    mm = lax.dot_general(
        kp.astype(jnp.bfloat16), xyz.astype(jnp.bfloat16),
        (((0,), (0,)), ((), ())),
        preferred_element_type=jnp.float32)
    sn = (kp[0] * kp[0] + kp[1] * kp[1]) + kp[2] * kp[2]
    dn = (xyz[0] * xyz[0] + xyz[1] * xyz[1]) + xyz[2] * xyz[2]
    d_ref[0] = (-2.0 * mm + sn[:, None]) + dn[None, :]


@jax.jit
def _dist(key_point, vertices):
    return pl.pallas_call(
        _dist_body,
        grid=(_B, _NP // _MROWS),
        in_specs=[
            pl.BlockSpec((1, 3, _MROWS), lambda b, j: (b, 0, j)),
            pl.BlockSpec((1, 3, _N), lambda b, j: (b, 0, 0)),
        ],
        out_specs=pl.BlockSpec((1, _MROWS, _N), lambda b, j: (b, j, 0)),
        out_shape=jax.ShapeDtypeStruct((_B, _NP, _N), jnp.float32),
    )(key_point, vertices)


_ROWS_W = _B * _NP // _NW
_CHUNKS = _N // _L


def _splat(ch, j):
    return lax.gather(
        ch, jnp.full((_L, 1), j, jnp.int32),
        lax.GatherDimensionNumbers(offset_dims=(), collapsed_slice_dims=(0,),
                                   start_index_map=(0,)),
        slice_sizes=(1,), mode=lax.GatherScatterMode.PROMISE_IN_BOUNDS)


_G = 16
_NG = _CHUNKS // _G


def _knn_body(d_hbm, vert_hbm, kp_hbm, out_hbm, prx_hbm, pry_hbm, prz_hbm,
              xv, yv, zv, kxv, kyv, kzv, idxbuf, pxb, pyb, pzb,
              db0, db1, sem0, sem1):
    wid = lax.axis_index("s") * _NC + lax.axis_index("c")
    b = wid // (_NW // _B)
    q = wid % (_NW // _B)
    m0 = q * _ROWS_W
    pltpu.sync_copy(vert_hbm.at[pl.ds((b * 3 + 0) * _N, _N)], xv)
    pltpu.sync_copy(vert_hbm.at[pl.ds((b * 3 + 1) * _N, _N)], yv)
    pltpu.sync_copy(vert_hbm.at[pl.ds((b * 3 + 2) * _N, _N)], zv)
    pltpu.sync_copy(kp_hbm.at[pl.ds((b * 3 + 0) * _NP + m0, _ROWS_W)], kxv)
    pltpu.sync_copy(kp_hbm.at[pl.ds((b * 3 + 1) * _NP + m0, _ROWS_W)], kyv)
    pltpu.sync_copy(kp_hbm.at[pl.ds((b * 3 + 2) * _NP + m0, _ROWS_W)], kzv)

    lane = lax.iota(jnp.int32, _L)
    row0 = b * _NP + m0

    def chunk_merge(d, cidx, carry):
        rv, ri, thr = carry
        hit = jnp.any(d < thr)

        def do_merge(args):
            rv_, ri_, _ = args
            sv, si = plsc.sort_key_val(d, cidx)
            rvr = lax.rev(rv_, (0,))
            rir = lax.rev(ri_, (0,))
            takea = sv < rvr
            lo_v = jnp.where(takea, sv, rvr)
            lo_i = jnp.where(takea, si, rir)
            nrv, nri = plsc.sort_key_val(lo_v, lo_i)
            return nrv, nri, _splat(nrv, 15)

        return lax.cond(hit, do_merge, lambda a: a, (rv, ri, thr))

    def do_row(r, dbuf):
        g, j = r // _L, r % _L
        kx = _splat(kxv[pl.ds(g * _L, _L)], j)
        ky = _splat(kyv[pl.ds(g * _L, _L)], j)
        kz = _splat(kzv[pl.ds(g * _L, _L)], j)

        def group_loop(t, carry):
            rv, ri, thr = carry
            base = t * _G * _L
            ds_ = [dbuf[pl.ds(base + u * _L, _L)] for u in range(_G)]
            gmin = ds_[0]
            for u in range(1, _G):
                gmin = jnp.minimum(gmin, ds_[u])
            ghit = jnp.any(gmin < thr)

            def scan_group(args):
                c = args
                for u in range(_G):
                    c = chunk_merge(ds_[u], lane + (base + u * _L), c)
                return c

            return lax.cond(ghit, scan_group, lambda a: a, (rv, ri, thr))

        rv0 = jnp.full((_L,), jnp.inf, jnp.float32)
        ri0 = jnp.zeros((_L,), jnp.int32)
        _, ri, _ = lax.fori_loop(0, _NG, group_loop, (rv0, ri0, rv0))
        idxbuf[pl.ds(r * _K, _K)] = ri + b * _N
        pxb[pl.ds(r * _K, _K)] = kx - plsc.load_gather(xv, [ri])
        pyb[pl.ds(r * _K, _K)] = ky - plsc.load_gather(yv, [ri])
        pzb[pl.ds(r * _K, _K)] = kz - plsc.load_gather(zv, [ri])

    pltpu.make_async_copy(d_hbm.at[pl.ds(row0 * _N, _N)], db0, sem0).start()

    def pair_loop(p, _):
        for par in range(2):
            r = 2 * p + par
            cur, csem = (db0, sem0) if par == 0 else (db1, sem1)
            nxt, nsem = (db1, sem1) if par == 0 else (db0, sem0)
            nr = jnp.minimum(r + 1, _ROWS_W - 1)
            pltpu.make_async_copy(
                d_hbm.at[pl.ds((row0 + nr) * _N, _N)], nxt, nsem).start()
            pltpu.make_async_copy(
                d_hbm.at[pl.ds((row0 + r) * _N, _N)], cur, csem).wait()
            do_row(r, cur)
        return 0

    lax.fori_loop(0, _ROWS_W // 2, pair_loop, 0)
    pltpu.make_async_copy(d_hbm.at[pl.ds(row0 * _N, _N)], db0, sem0).wait()

    o0 = (b * _NP + m0) * _K
    pltpu.sync_copy(idxbuf, out_hbm.at[pl.ds(o0, _ROWS_W * _K)])
    pltpu.sync_copy(pxb, prx_hbm.at[pl.ds(o0, _ROWS_W * _K)])
    pltpu.sync_copy(pyb, pry_hbm.at[pl.ds(o0, _ROWS_W * _K)])
    pltpu.sync_copy(pzb, prz_hbm.at[pl.ds(o0, _ROWS_W * _K)])


@jax.jit
def _knn(vertices, key_point):
    f = functools.partial(
        pl.kernel,
        out_type=[
            jax.ShapeDtypeStruct((_B * _NP * _K,), jnp.int32),
            jax.ShapeDtypeStruct((_B * _NP * _K,), jnp.float32),
            jax.ShapeDtypeStruct((_B * _NP * _K,), jnp.float32),
            jax.ShapeDtypeStruct((_B * _NP * _K,), jnp.float32),
        ],
        mesh=plsc.VectorSubcoreMesh(core_axis_name="c", subcore_axis_name="s"),
        compiler_params=pltpu.CompilerParams(needs_layout_passes=False),
        scratch_types=[
            pltpu.VMEM((_N,), jnp.float32),
            pltpu.VMEM((_N,), jnp.float32),
            pltpu.VMEM((_N,), jnp.float32),
            pltpu.VMEM((_ROWS_W,), jnp.float32),
            pltpu.VMEM((_ROWS_W,), jnp.float32),
            pltpu.VMEM((_ROWS_W,), jnp.float32),
            pltpu.VMEM((_ROWS_W * _K,), jnp.int32),
            pltpu.VMEM((_ROWS_W * _K,), jnp.float32),
            pltpu.VMEM((_ROWS_W * _K,), jnp.float32),
            pltpu.VMEM((_ROWS_W * _K,), jnp.float32),
            pltpu.VMEM((_N,), jnp.float32),
            pltpu.VMEM((_N,), jnp.float32),
            pltpu.SemaphoreType.DMA,
            pltpu.SemaphoreType.DMA,
        ],
    )(_knn_body)
    d = _dist(key_point, vertices)
    return f(d.reshape(-1), vertices.reshape(-1), key_point.reshape(-1))


_GCH = 128
_GN = _B * _NP * _K // _NW // _GCH


def _gather_body(tab_hbm, gidx_hbm, kidx_hbm, gout_hbm, kout_hbm,
                 idxv, rowsv, sem):
    wid = lax.axis_index("s") * _NC + lax.axis_index("c")
    base = wid * _GN * _GCH

    def chunk(t, _):
        off = base + t * _GCH
        pltpu.sync_copy(gidx_hbm.at[pl.ds(off, _GCH)], idxv)
        cp = pltpu.make_async_copy(tab_hbm.at[idxv], rowsv, sem)
        cp.start()
        cp.wait()
        pltpu.sync_copy(rowsv, gout_hbm.at[pl.ds(off, _GCH)])
        return 0

    lax.fori_loop(0, _GN, chunk, 0)

    kbase = wid * _GCH
    pltpu.sync_copy(kidx_hbm.at[pl.ds(kbase, _GCH)], idxv)
    cp = pltpu.make_async_copy(tab_hbm.at[idxv], rowsv, sem)
    cp.start()
    cp.wait()
    pltpu.sync_copy(rowsv, kout_hbm.at[pl.ds(kbase, _GCH)])


@jax.jit
def _gather(tab, gidx, kidx):
    f = functools.partial(
        pl.kernel,
        out_type=[
            jax.ShapeDtypeStruct((_B * _NP * _K, _C), jnp.float32),
            jax.ShapeDtypeStruct((_B * _NP, _C), jnp.float32),
        ],
        mesh=plsc.VectorSubcoreMesh(core_axis_name="c", subcore_axis_name="s"),
        compiler_params=pltpu.CompilerParams(needs_layout_passes=False),
        scratch_types=[
            pltpu.VMEM((_GCH,), jnp.int32),
            pltpu.VMEM((_GCH, _C), jnp.float32),
            pltpu.SemaphoreType.DMA,
        ],
    )(_gather_body)
    return f(tab, gidx, kidx)


def _mlp_body(g_ref, kf_ref, px_ref, py_ref, pz_ref,
              w1_ref, b1_ref, w2_ref, b2_ref,
              a1_ref, ab1_ref, a2_ref, ab2_ref, out_ref):
    feat = g_ref[...]
    kf = kf_ref[...]
    kfe = jnp.broadcast_to(kf[:, None, :], (_MB, _K, _C)).reshape(_RB, _C)

    dot = functools.partial(jnp.dot, preferred_element_type=jnp.float32)
    px, py, pz = px_ref[...], py_ref[...], pz_ref[...]
    h = (px * w1_ref[0:1, :] + py * w1_ref[1:2, :] + pz * w1_ref[2:3, :]
         + b1_ref[...])
    h = jnp.where(h >= 0, h, 0.2 * h)
    pe = dot(h, w2_ref[...]) + b2_ref[...]
    qk = kfe - feat
    a = dot(qk + pe, a1_ref[...]) + ab1_ref[...]
    a = jnp.where(a >= 0, a, 0.2 * a)
    logits = dot(a, a2_ref[...]) + ab2_ref[...]

    l3 = logits.reshape(_MB, _K, _C)
    mx = jnp.max(l3, axis=1, keepdims=True)
    e = jnp.exp(l3 - mx)
    w = e / jnp.sum(e, axis=1, keepdims=True)
    v3 = (feat + pe).reshape(_MB, _K, _C)
    out_ref[...] = jnp.sum(w * v3, axis=1)


@jax.jit
def _mlp(g_rows, k_rows, px, py, pz, w1, b1, w2, b2, a1, ab1, a2, ab2):
    nblk = _B * _NP // _MB
    wspec = lambda shp: pl.BlockSpec(shp, lambda i: (0, 0))
    return pl.pallas_call(
        _mlp_body,
        grid=(nblk,),
        in_specs=[
            pl.BlockSpec((_RB, _C), lambda i: (i, 0)),
            pl.BlockSpec((_MB, _C), lambda i: (i, 0)),
            pl.BlockSpec((_RB, 1), lambda i: (i, 0)),
            pl.BlockSpec((_RB, 1), lambda i: (i, 0)),
            pl.BlockSpec((_RB, 1), lambda i: (i, 0)),
            wspec((8, _D)), wspec((1, _D)),
            wspec((_D, _C)), wspec((1, _C)),
            wspec((_C, _D)), wspec((1, _D)),
            wspec((_D, _C)), wspec((1, _C)),
        ],
        out_specs=pl.BlockSpec((_MB, _C), lambda i: (i, 0)),
        out_shape=jax.ShapeDtypeStruct((_B * _NP, _C), jnp.float32),
    )(g_rows, k_rows, px, py, pz, w1, b1, w2, b2, a1, ab1, a2, ab2)


def kernel(vertices, feature_map, pW1, pb1, pg1, pbe1, pW2, pb2,
           aW1, ab1, ag1, abe1, aW2, ab2):
    key_point, fps_g = _fps(vertices)
    knn_g, prx, pry, prz = _knn(vertices, key_point)

    feat_t = jnp.transpose(feature_map, (0, 2, 1)).reshape(_B * _N, _C)
    g_rows, k_rows = _gather(feat_t, knn_g, fps_g.reshape(-1))

    inv = jnp.float32(1.0) / jnp.sqrt(jnp.float32(1.0 + 1e-5))
    ps, asc = pg1 * inv, ag1 * inv
    w1 = jnp.pad(pW1, ((0, 0), (0, 8 - 3))).T * ps[None, :]
    b1 = (pb1 * ps + pbe1)[None, :]
    w2 = pW2.T
    b2 = pb2[None, :]
    a1 = aW1.T * asc[None, :]
    ab1f = (ab1 * asc + abe1)[None, :]
    a2 = aW2.T
    ab2f = ab2[None, :]

    out = _mlp(g_rows, k_rows, prx[:, None], pry[:, None], prz[:, None],
               w1, b1, w2, b2, a1, ab1f, a2, ab2f)
    new_feat = jnp.transpose(out.reshape(_B, _NP, _C), (0, 2, 1))
    return key_point, new_feat


# speedup vs baseline: 6.9964x; 1.0029x over previous
import functools

import jax
import jax.numpy as jnp
from jax import lax
from jax.experimental import pallas as pl
from jax.experimental.pallas import tpu as pltpu
from jax.experimental.pallas import tpu_sc as plsc

_B, _N, _C, _NP, _K, _D = 8, 4096, 256, 512, 16, 64
_L = 16
_NC, _NS = 2, 16
_NW = _NC * _NS
_MB = 32
_RB = _MB * _K


def _fps_body(v_ref, kp_ref, fpsg_ref, dist_ref):
    iota_p = lax.broadcasted_iota(jnp.int32, (_B, _NP), 1)
    boff = lax.broadcasted_iota(jnp.int32, (_B, 1), 0) * _N
    dist_ref[...] = v_ref[:, 0, :] * 0.0 + 1e10

    def body(i, st):
        far, kx, ky, kz, fi = st
        iota_n = lax.broadcasted_iota(jnp.int32, (_B, _N), 1)
        onehot = iota_n == far
        x = v_ref[:, 0, :]
        y = v_ref[:, 1, :]
        z = v_ref[:, 2, :]
        cx = jnp.sum(jnp.where(onehot, x, 0.0), axis=1, keepdims=True)
        cy = jnp.sum(jnp.where(onehot, y, 0.0), axis=1, keepdims=True)
        cz = jnp.sum(jnp.where(onehot, z, 0.0), axis=1, keepdims=True)
        sel = iota_p == i
        kx = jnp.where(sel, jnp.broadcast_to(cx, (_B, _NP)), kx)
        ky = jnp.where(sel, jnp.broadcast_to(cy, (_B, _NP)), ky)
        kz = jnp.where(sel, jnp.broadcast_to(cz, (_B, _NP)), kz)
        fi = jnp.where(sel, jnp.broadcast_to(far, (_B, _NP)), fi)
        dx, dy, dz = x - cx, y - cy, z - cz
        d = (dx * dx + dy * dy) + dz * dz
        dists = jnp.minimum(dist_ref[...], d)
        dist_ref[...] = dists
        m = jnp.max(dists, axis=1, keepdims=True)
        far = jnp.min(jnp.where(dists == m, iota_n, _N), axis=1, keepdims=True)
        return far, kx, ky, kz, fi

    init = (
        jnp.zeros((_B, 1), jnp.int32),
        v_ref[:, 0, :_NP],
        v_ref[:, 1, :_NP],
        v_ref[:, 2, :_NP],
        iota_p + boff,
    )
    _, kx, ky, kz, fi = lax.fori_loop(0, _NP, body, init)
    kp_ref[:, 0, :] = kx
    kp_ref[:, 1, :] = ky
    kp_ref[:, 2, :] = kz
    fpsg_ref[...] = fi + boff


@jax.jit
def _fps(vertices):
    return pl.pallas_call(
        _fps_body,
        out_shape=[
            jax.ShapeDtypeStruct((_B, 3, _NP), jnp.float32),
            jax.ShapeDtypeStruct((_B, _NP), jnp.int32),
        ],
        scratch_shapes=[pltpu.VMEM((_B, _N), jnp.float32)],
    )(vertices)


_MROWS = 256


def _dist_body(kp_ref, v_ref, d_ref):
    kp = kp_ref[0]
    xyz = v_ref[0]
    mm = lax.dot_general(
        kp.astype(jnp.bfloat16), xyz.astype(jnp.bfloat16),
        (((0,), (0,)), ((), ())),
        preferred_element_type=jnp.float32)
    sn = (kp[0] * kp[0] + kp[1] * kp[1]) + kp[2] * kp[2]
    dn = (xyz[0] * xyz[0] + xyz[1] * xyz[1]) + xyz[2] * xyz[2]
    d_ref[0] = (-2.0 * mm + sn[:, None]) + dn[None, :]


@jax.jit
def _dist(key_point, vertices):
    return pl.pallas_call(
        _dist_body,
        grid=(_B, _NP // _MROWS),
        in_specs=[
            pl.BlockSpec((1, 3, _MROWS), lambda b, j: (b, 0, j)),
            pl.BlockSpec((1, 3, _N), lambda b, j: (b, 0, 0)),
        ],
        out_specs=pl.BlockSpec((1, _MROWS, _N), lambda b, j: (b, j, 0)),
        out_shape=jax.ShapeDtypeStruct((_B, _NP, _N), jnp.float32),
    )(key_point, vertices)


_ROWS_W = _B * _NP // _NW
_CHUNKS = _N // _L


def _splat(ch, j):
    return lax.gather(
        ch, jnp.full((_L, 1), j, jnp.int32),
        lax.GatherDimensionNumbers(offset_dims=(), collapsed_slice_dims=(0,),
                                   start_index_map=(0,)),
        slice_sizes=(1,), mode=lax.GatherScatterMode.PROMISE_IN_BOUNDS)


_G = 16
_NG = _CHUNKS // _G


def _knn_body(d_hbm, vert_hbm, kp_hbm, out_hbm, prx_hbm, pry_hbm, prz_hbm,
              xv, yv, zv, kxv, kyv, kzv, idxbuf, pxb, pyb, pzb,
              db0, db1, sem0, sem1):
    wid = lax.axis_index("s") * _NC + lax.axis_index("c")
    b = wid // (_NW // _B)
    q = wid % (_NW // _B)
    m0 = q * _ROWS_W
    pltpu.sync_copy(vert_hbm.at[pl.ds((b * 3 + 0) * _N, _N)], xv)
    pltpu.sync_copy(vert_hbm.at[pl.ds((b * 3 + 1) * _N, _N)], yv)
    pltpu.sync_copy(vert_hbm.at[pl.ds((b * 3 + 2) * _N, _N)], zv)
    pltpu.sync_copy(kp_hbm.at[pl.ds((b * 3 + 0) * _NP + m0, _ROWS_W)], kxv)
    pltpu.sync_copy(kp_hbm.at[pl.ds((b * 3 + 1) * _NP + m0, _ROWS_W)], kyv)
    pltpu.sync_copy(kp_hbm.at[pl.ds((b * 3 + 2) * _NP + m0, _ROWS_W)], kzv)

    lane = lax.iota(jnp.int32, _L)
    row0 = b * _NP + m0

    def chunk_merge(d, cidx, carry):
        rv, ri, thr = carry
        hit = jnp.any(d < thr)

        def do_merge(args):
            rv_, ri_, _ = args
            sv, si = plsc.sort_key_val(d, cidx)
            rvr = lax.rev(rv_, (0,))
            rir = lax.rev(ri_, (0,))
            takea = sv < rvr
            lo_v = jnp.where(takea, sv, rvr)
            lo_i = jnp.where(takea, si, rir)
            nrv, nri = plsc.sort_key_val(lo_v, lo_i)
            return nrv, nri, _splat(nrv, 15)

        return lax.cond(hit, do_merge, lambda a: a, (rv, ri, thr))

    def do_row(r, dbuf):
        g, j = r // _L, r % _L
        kx = _splat(kxv[pl.ds(g * _L, _L)], j)
        ky = _splat(kyv[pl.ds(g * _L, _L)], j)
        kz = _splat(kzv[pl.ds(g * _L, _L)], j)

        def group_loop(t, carry):
            rv, ri, thr = carry
            base = t * _G * _L
            ds_ = [dbuf[pl.ds(base + u * _L, _L)] for u in range(_G)]
            gmin = ds_[0]
            for u in range(1, _G):
                gmin = jnp.minimum(gmin, ds_[u])
            ghit = jnp.any(gmin < thr)

            def scan_group(args):
                c = args
                for u in range(_G):
                    c = chunk_merge(ds_[u], lane + (base + u * _L), c)
                return c

            return lax.cond(ghit, scan_group, lambda a: a, (rv, ri, thr))

        rv0 = jnp.full((_L,), jnp.inf, jnp.float32)
        ri0 = jnp.zeros((_L,), jnp.int32)
        _, ri, _ = lax.fori_loop(0, _NG, group_loop, (rv0, ri0, rv0))
        idxbuf[pl.ds(r * _K, _K)] = ri + b * _N
        pxb[pl.ds(r * _K, _K)] = kx - plsc.load_gather(xv, [ri])
        pyb[pl.ds(r * _K, _K)] = ky - plsc.load_gather(yv, [ri])
        pzb[pl.ds(r * _K, _K)] = kz - plsc.load_gather(zv, [ri])

    pltpu.make_async_copy(d_hbm.at[pl.ds(row0 * _N, _N)], db0, sem0).start()

    def pair_loop(p, _):
        for par in range(2):
            r = 2 * p + par
            cur, csem = (db0, sem0) if par == 0 else (db1, sem1)
            nxt, nsem = (db1, sem1) if par == 0 else (db0, sem0)
            nr = jnp.minimum(r + 1, _ROWS_W - 1)
            pltpu.make_async_copy(
                d_hbm.at[pl.ds((row0 + nr) * _N, _N)], nxt, nsem).start()
            pltpu.make_async_copy(
                d_hbm.at[pl.ds((row0 + r) * _N, _N)], cur, csem).wait()
            do_row(r, cur)
        return 0

    lax.fori_loop(0, _ROWS_W // 2, pair_loop, 0)
    pltpu.make_async_copy(d_hbm.at[pl.ds(row0 * _N, _N)], db0, sem0).wait()

    o0 = (b * _NP + m0) * _K
    pltpu.sync_copy(idxbuf, out_hbm.at[pl.ds(o0, _ROWS_W * _K)])
    pltpu.sync_copy(pxb, prx_hbm.at[pl.ds(o0, _ROWS_W * _K)])
    pltpu.sync_copy(pyb, pry_hbm.at[pl.ds(o0, _ROWS_W * _K)])
    pltpu.sync_copy(pzb, prz_hbm.at[pl.ds(o0, _ROWS_W * _K)])


@jax.jit
def _knn(vertices, key_point):
    f = functools.partial(
        pl.kernel,
        out_type=[
            jax.ShapeDtypeStruct((_B * _NP * _K,), jnp.int32),
            jax.ShapeDtypeStruct((_B * _NP * _K,), jnp.float32),
            jax.ShapeDtypeStruct((_B * _NP * _K,), jnp.float32),
            jax.ShapeDtypeStruct((_B * _NP * _K,), jnp.float32),
        ],
        mesh=plsc.VectorSubcoreMesh(core_axis_name="c", subcore_axis_name="s"),
        compiler_params=pltpu.CompilerParams(needs_layout_passes=False),
        scratch_types=[
            pltpu.VMEM((_N,), jnp.float32),
            pltpu.VMEM((_N,), jnp.float32),
            pltpu.VMEM((_N,), jnp.float32),
            pltpu.VMEM((_ROWS_W,), jnp.float32),
            pltpu.VMEM((_ROWS_W,), jnp.float32),
            pltpu.VMEM((_ROWS_W,), jnp.float32),
            pltpu.VMEM((_ROWS_W * _K,), jnp.int32),
            pltpu.VMEM((_ROWS_W * _K,), jnp.float32),
            pltpu.VMEM((_ROWS_W * _K,), jnp.float32),
            pltpu.VMEM((_ROWS_W * _K,), jnp.float32),
            pltpu.VMEM((_N,), jnp.float32),
            pltpu.VMEM((_N,), jnp.float32),
            pltpu.SemaphoreType.DMA,
            pltpu.SemaphoreType.DMA,
        ],
    )(_knn_body)
    d = _dist(key_point, vertices)
    return f(d.reshape(-1), vertices.reshape(-1), key_point.reshape(-1))


_GCH = 128
_GN = _B * _NP * _K // _NW // _GCH


def _gather_body(tab_hbm, gidx_hbm, kidx_hbm, gout_hbm, kout_hbm,
                 idxv, rowsv, sem):
    wid = lax.axis_index("s") * _NC + lax.axis_index("c")
    base = wid * _GN * _GCH

    def chunk(t, _):
        off = base + t * _GCH
        pltpu.sync_copy(gidx_hbm.at[pl.ds(off, _GCH)], idxv)
        cp = pltpu.make_async_copy(tab_hbm.at[idxv], rowsv, sem)
        cp.start()
        cp.wait()
        pltpu.sync_copy(rowsv, gout_hbm.at[pl.ds(off, _GCH)])
        return 0

    lax.fori_loop(0, _GN, chunk, 0)

    kbase = wid * _GCH
    pltpu.sync_copy(kidx_hbm.at[pl.ds(kbase, _GCH)], idxv)
    cp = pltpu.make_async_copy(tab_hbm.at[idxv], rowsv, sem)
    cp.start()
    cp.wait()
    pltpu.sync_copy(rowsv, kout_hbm.at[pl.ds(kbase, _GCH)])


@jax.jit
def _gather(tab, gidx, kidx):
    f = functools.partial(
        pl.kernel,
        out_type=[
            jax.ShapeDtypeStruct((_B * _NP * _K, _C), jnp.float32),
            jax.ShapeDtypeStruct((_B * _NP, _C), jnp.float32),
        ],
        mesh=plsc.VectorSubcoreMesh(core_axis_name="c", subcore_axis_name="s"),
        compiler_params=pltpu.CompilerParams(needs_layout_passes=False),
        scratch_types=[
            pltpu.VMEM((_GCH,), jnp.int32),
            pltpu.VMEM((_GCH, _C), jnp.float32),
            pltpu.SemaphoreType.DMA,
        ],
    )(_gather_body)
    return f(tab, gidx, kidx)


def _mlp_body(g_ref, kf_ref, px_ref, py_ref, pz_ref,
              w1_ref, b1_ref, w2_ref, b2_ref,
              a1_ref, ab1_ref, a2_ref, ab2_ref, out_ref):
    feat = g_ref[...]
    kf = kf_ref[...]
    kfe = jnp.broadcast_to(kf[:, None, :], (_MB, _K, _C)).reshape(_RB, _C)

    dot = functools.partial(jnp.dot, preferred_element_type=jnp.float32)
    px, py, pz = px_ref[...], py_ref[...], pz_ref[...]
    h = (px * w1_ref[0:1, :] + py * w1_ref[1:2, :] + pz * w1_ref[2:3, :]
         + b1_ref[...])
    h = jnp.where(h >= 0, h, 0.2 * h)
    pe = dot(h, w2_ref[...]) + b2_ref[...]
    qk = kfe - feat
    a = dot(qk + pe, a1_ref[...]) + ab1_ref[...]
    a = jnp.where(a >= 0, a, 0.2 * a)
    logits = dot(a, a2_ref[...]) + ab2_ref[...]

    l3 = logits.reshape(_MB, _K, _C)
    mx = jnp.max(l3, axis=1, keepdims=True)
    e = jnp.exp(l3 - mx)
    w = e / jnp.sum(e, axis=1, keepdims=True)
    v3 = (feat + pe).reshape(_MB, _K, _C)
    out_ref[...] = jnp.sum(w * v3, axis=1)


@jax.jit
def _mlp(g_rows, k_rows, px, py, pz, w1, b1, w2, b2, a1, ab1, a2, ab2):
    nblk = _B * _NP // _MB
    wspec = lambda shp: pl.BlockSpec(shp, lambda i: (0, 0))
    return pl.pallas_call(
        _mlp_body,
        grid=(nblk,),
        in_specs=[
            pl.BlockSpec((_RB, _C), lambda i: (i, 0)),
            pl.BlockSpec((_MB, _C), lambda i: (i, 0)),
            pl.BlockSpec((_RB, 1), lambda i: (i, 0)),
            pl.BlockSpec((_RB, 1), lambda i: (i, 0)),
            pl.BlockSpec((_RB, 1), lambda i: (i, 0)),
            wspec((8, _D)), wspec((1, _D)),
            wspec((_D, _C)), wspec((1, _C)),
            wspec((_C, _D)), wspec((1, _D)),
            wspec((_D, _C)), wspec((1, _C)),
        ],
        out_specs=pl.BlockSpec((_MB, _C), lambda i: (i, 0)),
        out_shape=jax.ShapeDtypeStruct((_B * _NP, _C), jnp.float32),
    )(g_rows, k_rows, px, py, pz, w1, b1, w2, b2, a1, ab1, a2, ab2)


def kernel(vertices, feature_map, pW1, pb1, pg1, pbe1, pW2, pb2,
           aW1, ab1, ag1, abe1, aW2, ab2):
    key_point, fps_g = _fps(vertices)
    knn_g, prx, pry, prz = _knn(vertices, key_point)

    feat_t = jnp.transpose(feature_map, (0, 2, 1)).reshape(_B * _N, _C)
    g_rows, k_rows = _gather(feat_t, knn_g, fps_g.reshape(-1))

    inv = jnp.float32(1.0) / jnp.sqrt(jnp.float32(1.0 + 1e-5))
    ps, asc = pg1 * inv, ag1 * inv
    w1 = jnp.pad(pW1, ((0, 0), (0, 8 - 3))).T * ps[None, :]
    b1 = (pb1 * ps + pbe1)[None, :]
    w2 = pW2.T
    b2 = pb2[None, :]
    a1 = aW1.T * asc[None, :]
    ab1f = (ab1 * asc + abe1)[None, :]
    a2 = aW2.T
    ab2f = ab2[None, :]

    out = _mlp(g_rows, k_rows, prx[:, None], pry[:, None], prz[:, None],
               w1, b1, w2, b2, a1, ab1f, a2, ab2f)
    new_feat = jnp.transpose(out.reshape(_B, _NP, _C), (0, 2, 1))
    return key_point, new_feat
